# baseline (speedup 1.0000x reference)
import functools

import jax
import jax.numpy as jnp
from jax import lax
from jax.experimental import pallas as pl
from jax.experimental.pallas import tpu as pltpu
from jax.experimental.pallas import tpu_sc as plsc

N = 6000
E = 96000
D = 128
C = 10
NPAD = 6144

NC = 2
NS = 16
NW = NC * NS
EP = 98304
NEPW = EP // NW
KS = 128
NCHB = NEPW // KS
NBUF = 3
NSC = 6016
RPT = NSC // NS
CNTW = 16

BN = 1000
BR = 512
CB = 512


def _sc_agg_body(feat, srcm, dstm, zfeat, zcnt, ones_h, psum, pcnt,
                 acc, cnta, sidx, didx, rows0, rows1, rows2, ones_v,
                 sem0, sem1, sem2):
    c = lax.axis_index("c")
    s = lax.axis_index("s")
    wid = s * NC + c
    base = s * RPT
    pltpu.sync_copy(zfeat, acc.at[pl.ds(base, RPT)])
    pltpu.sync_copy(zcnt, cnta.at[pl.ds(base, RPT)])
    pltpu.sync_copy(srcm.at[wid], sidx)
    pltpu.sync_copy(dstm.at[wid], didx)
    pltpu.sync_copy(ones_h, ones_v)
    plsc.subcore_barrier()

    bufs = [rows0, rows1, rows2]
    sems = [sem0, sem1, sem2]

    descs = [None] * NBUF
    for g in range(NBUF):
        descs[g] = pltpu.async_copy(feat.at[sidx.at[g]], bufs[g], sems[g])
    for b in range(NCHB):
        r = b % NBUF
        descs[r].wait()
        d = didx.at[b]
        pltpu.sync_copy(bufs[r], acc.at[d], add=True)
        pltpu.sync_copy(ones_v, cnta.at[d], add=True)
        if b + NBUF < NCHB:
            descs[r] = pltpu.async_copy(
                feat.at[sidx.at[b + NBUF]], bufs[r], sems[r])

    plsc.subcore_barrier()
    pltpu.sync_copy(acc.at[pl.ds(base, RPT)], psum.at[c, pl.ds(base, RPT)])
    pltpu.sync_copy(cnta.at[pl.ds(base, RPT)], pcnt.at[c, pl.ds(base, RPT)])


@functools.cache
def _get_sc_agg():
    return pl.kernel(
        _sc_agg_body,
        out_type=(jax.ShapeDtypeStruct((NC, NSC, D), jnp.float32),
                  jax.ShapeDtypeStruct((NC, NSC, CNTW), jnp.float32)),
        mesh=plsc.VectorSubcoreMesh(core_axis_name="c", subcore_axis_name="s"),
        scratch_types=[
            pltpu.VMEM_SHARED((NSC, D), jnp.float32),
            pltpu.VMEM_SHARED((NSC, CNTW), jnp.float32),
            pltpu.VMEM((NCHB, KS), jnp.int32),
            pltpu.VMEM((NCHB, KS), jnp.int32),
            pltpu.VMEM((KS, D), jnp.float32),
            pltpu.VMEM((KS, D), jnp.float32),
            pltpu.VMEM((KS, D), jnp.float32),
            pltpu.VMEM((KS, CNTW), jnp.float32),
            pltpu.SemaphoreType.DMA,
            pltpu.SemaphoreType.DMA,
            pltpu.SemaphoreType.DMA,
        ],
    )


def _sage_body(p, cn, xb, wlt, wrt, b, h_ref, *, relu):
    cb = cn[...]
    cnt = jnp.maximum(cb[0][:, :1] + cb[1][:, :1], 1.0)
    pb = p[...]
    agg = (pb[0] + pb[1]) / cnt
    y = jnp.dot(agg, wlt[...], preferred_element_type=jnp.float32)
    y = y + jnp.dot(xb[...], wrt[...], preferred_element_type=jnp.float32)
    y = y + b[...]
    if relu:
        y = jnp.maximum(y, 0.0)
    h_ref[...] = y


def _sage2_body(p, cn, xb, wlt, wrt, b, auxc, h_ref, hd_ref, hn_ref):
    cb = cn[...]
    cnt = jnp.maximum(cb[0][:, :1] + cb[1][:, :1], 1.0)
    pb = p[...]
    agg = (pb[0] + pb[1]) / cnt
    y = jnp.dot(agg, wlt[...], preferred_element_type=jnp.float32)
    y = y + jnp.dot(xb[...], wrt[...], preferred_element_type=jnp.float32)
    y = y + b[...]
    h_ref[...] = y
    hd_ref[...] = jnp.dot(y, auxc[...], preferred_element_type=jnp.float32)
    hn_ref[...] = jnp.sum(y * y, axis=1, keepdims=True)


_row_spec = pl.BlockSpec((BN, D), lambda i: (i, 0))
_psum_spec = pl.BlockSpec((NC, BN, D), lambda i: (0, i, 0))
_pcnt_spec = pl.BlockSpec((NC, BN, CNTW), lambda i: (0, i, 0))
_full_dd = pl.BlockSpec((D, D), lambda i: (0, 0))
_bias_spec = pl.BlockSpec((1, D), lambda i: (0, 0))

_sage1 = pl.pallas_call(
    functools.partial(_sage_body, relu=True),
    grid=(N // BN,),
    in_specs=[_psum_spec, _pcnt_spec, _row_spec,
              _full_dd, _full_dd, _bias_spec],
    out_specs=_row_spec,
    out_shape=jax.ShapeDtypeStruct((N, D), jnp.float32),
)

_sage2 = pl.pallas_call(
    _sage2_body,
    grid=(N // BN,),
    in_specs=[_psum_spec, _pcnt_spec, _row_spec,
              _full_dd, _full_dd, _bias_spec,
              pl.BlockSpec((D, 1), lambda i: (0, 0))],
    out_specs=(_row_spec,
               pl.BlockSpec((BN, 1), lambda i: (i, 0)),
               pl.BlockSpec((BN, 1), lambda i: (i, 0))),
    out_shape=(jax.ShapeDtypeStruct((N, D), jnp.float32),
               jax.ShapeDtypeStruct((N, 1), jnp.float32),
               jax.ShapeDtypeStruct((N, 1), jnp.float32)),
)


def _fused_body(hp, hrow, sdr, snr, sdT, snT, auxr, w1, w2, bcr, out_ref):
    eps = 1e-8
    a = auxr[...]
    an = jnp.maximum(jnp.sqrt(jnp.sum(a * a)), eps)
    sr = sdr[...] / (jnp.maximum(jnp.sqrt(snr[...]), eps) * an)
    sc = sdT[...] / (jnp.maximum(jnp.sqrt(snT[...]), eps) * an)
    acc = jnp.zeros((BR, D), jnp.float32)
    rs = jnp.zeros((BR, 1), jnp.float32)
    for t in range(NPAD // CB):
        dd = sc[:, t * CB:(t + 1) * CB] - sr
        kap = jnp.exp(-(dd * dd))
        acc = acc + jnp.dot(kap, hp[t * CB:(t + 1) * CB, :],
                            preferred_element_type=jnp.float32)
        rs = rs + jnp.sum(kap, axis=1, keepdims=True)
    z = acc / rs
    o = jnp.dot(hrow[...], w1[...], preferred_element_type=jnp.float32)
    o = o + jnp.dot(z, w2[...], preferred_element_type=jnp.float32)
    o = o + bcr[...]
    out_ref[...] = o


_fused = pl.pallas_call(
    _fused_body,
    grid=(NPAD // BR,),
    in_specs=[pl.BlockSpec((NPAD, D), lambda i: (0, 0)),
              pl.BlockSpec((BR, D), lambda i: (i, 0)),
              pl.BlockSpec((BR, 1), lambda i: (i, 0)),
              pl.BlockSpec((BR, 1), lambda i: (i, 0)),
              pl.BlockSpec((1, NPAD), lambda i: (0, 0)),
              pl.BlockSpec((1, NPAD), lambda i: (0, 0)),
              pl.BlockSpec((1, D), lambda i: (0, 0)),
              pl.BlockSpec((D, C), lambda i: (0, 0)),
              pl.BlockSpec((D, C), lambda i: (0, 0)),
              pl.BlockSpec((1, C), lambda i: (0, 0))],
    out_specs=pl.BlockSpec((BR, C), lambda i: (i, 0)),
    out_shape=jax.ShapeDtypeStruct((NPAD, C), jnp.float32),
)


def kernel(x, edge_index, W_l1, b_l1, W_r1, W_l2, b_l2, W_r2, aux, Wc, bc):
    pad = jnp.zeros((EP - E,), jnp.int32)
    src = jnp.concatenate([edge_index[0], pad]).reshape(NW, NCHB, KS)
    dst = jnp.concatenate([edge_index[1], pad + N]).reshape(NW, NCHB, KS)
    zfeat = jnp.zeros((RPT, D), jnp.float32)
    zcnt = jnp.zeros((RPT, CNTW), jnp.float32)
    ones_h = jnp.ones((KS, CNTW), jnp.float32)

    sc_agg = _get_sc_agg()
    psum1, pcnt1 = sc_agg(x, src, dst, zfeat, zcnt, ones_h)
    h1 = _sage1(psum1, pcnt1, x, W_l1.T, W_r1.T, b_l1.reshape(1, D))

    psum2, pcnt2 = sc_agg(h1, src, dst, zfeat, zcnt, ones_h)
    h, hd, hn = _sage2(psum2, pcnt2, h1,
                       W_l2.T, W_r2.T, b_l2.reshape(1, D), aux.reshape(D, 1))

    hp = jnp.pad(h, ((0, NPAD - N), (0, 0)))
    sdr = jnp.pad(hd, ((0, NPAD - N), (0, 0)), constant_values=1e30)
    snr = jnp.pad(hn, ((0, NPAD - N), (0, 0)), constant_values=1.0)
    out = _fused(hp, hp, sdr, snr, sdr.reshape(1, NPAD), snr.reshape(1, NPAD),
                 aux.reshape(1, D), Wc[:, :D].T, Wc[:, D:].T, bc.reshape(1, C))
    return out[:N]


# speedup vs baseline: 6.7880x; 6.7880x over previous
import functools

import jax
import jax.numpy as jnp
from jax import lax
from jax.experimental import pallas as pl
from jax.experimental.pallas import tpu as pltpu
from jax.experimental.pallas import tpu_sc as plsc

N = 6000
E = 96000
D = 128
C = 10
NPAD = 6144

NC = 2
NS = 16
NW = NC * NS
EPW = E // NW
K = 120
NCH = EPW // K
NSC = 6144
RPT = NSC // NS

BN = 1000
BR = 512
CB = 512


def _sc_agg_body(feat, srcm, dstm, zfeat, ones_h, *refs, do_cnt):
    if do_cnt:
        psum, pcnt, acc, cnta, gidx, didx, rows, ones_v, gsem = refs
    else:
        psum, acc, gidx, didx, rows, gsem = refs
    c = lax.axis_index("c")
    s = lax.axis_index("s")
    wid = s * NC + c
    base = s * RPT
    pltpu.sync_copy(zfeat, acc.at[pl.ds(base, RPT)])
    if do_cnt:
        pltpu.sync_copy(zfeat, cnta.at[pl.ds(base, RPT)])
        pltpu.sync_copy(ones_h, ones_v)
    plsc.subcore_barrier()

    def body(j, carry):
        pltpu.sync_copy(srcm.at[wid, j], gidx)
        pltpu.sync_copy(dstm.at[wid, j], didx)
        pltpu.async_copy(feat.at[gidx], rows, gsem).wait()
        pltpu.sync_copy(rows, acc.at[didx], add=True)
        if do_cnt:
            pltpu.sync_copy(ones_v, cnta.at[didx], add=True)
        return carry

    lax.fori_loop(0, NCH, body, 0)
    plsc.subcore_barrier()
    pltpu.sync_copy(acc.at[pl.ds(base, RPT)], psum.at[c, pl.ds(base, RPT)])
    if do_cnt:
        pltpu.sync_copy(cnta.at[pl.ds(base, RPT)], pcnt.at[c, pl.ds(base, RPT)])


@functools.cache
def _get_sc_agg(do_cnt):
    psum_t = jax.ShapeDtypeStruct((NC, NSC, D), jnp.float32)
    out_type = (psum_t, psum_t) if do_cnt else psum_t
    scratch = [pltpu.VMEM_SHARED((NSC, D), jnp.float32)]
    if do_cnt:
        scratch.append(pltpu.VMEM_SHARED((NSC, D), jnp.float32))
    scratch += [
        pltpu.VMEM((K,), jnp.int32),
        pltpu.VMEM((K,), jnp.int32),
        pltpu.VMEM((K, D), jnp.float32),
    ]
    if do_cnt:
        scratch.append(pltpu.VMEM((K, D), jnp.float32))
    scratch.append(pltpu.SemaphoreType.DMA)
    return pl.kernel(
        functools.partial(_sc_agg_body, do_cnt=do_cnt),
        out_type=out_type,
        mesh=plsc.VectorSubcoreMesh(core_axis_name="c", subcore_axis_name="s"),
        scratch_types=scratch,
    )


def _sage_body(p, cn, xb, wlt, wrt, b, h_ref, *, relu):
    pb = p[...]
    cb = cn[...]
    cnt = jnp.maximum(cb[0][:, :1] + cb[1][:, :1], 1.0)
    agg = (pb[0] + pb[1]) / cnt
    y = jnp.dot(agg, wlt[...], preferred_element_type=jnp.float32)
    y = y + jnp.dot(xb[...], wrt[...], preferred_element_type=jnp.float32)
    y = y + b[...]
    if relu:
        y = jnp.maximum(y, 0.0)
    h_ref[...] = y


def _sage2_body(p, cn, xb, wlt, wrt, b, auxc, h_ref, hd_ref, hn_ref):
    pb = p[...]
    cb = cn[...]
    cnt = jnp.maximum(cb[0][:, :1] + cb[1][:, :1], 1.0)
    agg = (pb[0] + pb[1]) / cnt
    y = jnp.dot(agg, wlt[...], preferred_element_type=jnp.float32)
    y = y + jnp.dot(xb[...], wrt[...], preferred_element_type=jnp.float32)
    y = y + b[...]
    h_ref[...] = y
    hd_ref[...] = jnp.dot(y, auxc[...], preferred_element_type=jnp.float32)
    hn_ref[...] = jnp.sum(y * y, axis=1, keepdims=True)


_row_spec = pl.BlockSpec((BN, D), lambda i: (i, 0))
_psum_spec = pl.BlockSpec((NC, BN, D), lambda i: (0, i, 0))
_full_dd = pl.BlockSpec((D, D), lambda i: (0, 0))
_bias_spec = pl.BlockSpec((1, D), lambda i: (0, 0))

_sage1 = pl.pallas_call(
    functools.partial(_sage_body, relu=True),
    grid=(N // BN,),
    in_specs=[_psum_spec, _psum_spec, _row_spec,
              _full_dd, _full_dd, _bias_spec],
    out_specs=_row_spec,
    out_shape=jax.ShapeDtypeStruct((N, D), jnp.float32),
)

_sage2 = pl.pallas_call(
    _sage2_body,
    grid=(N // BN,),
    in_specs=[_psum_spec, _psum_spec, _row_spec,
              _full_dd, _full_dd, _bias_spec,
              pl.BlockSpec((D, 1), lambda i: (0, 0))],
    out_specs=(_row_spec,
               pl.BlockSpec((BN, 1), lambda i: (i, 0)),
               pl.BlockSpec((BN, 1), lambda i: (i, 0))),
    out_shape=(jax.ShapeDtypeStruct((N, D), jnp.float32),
               jax.ShapeDtypeStruct((N, 1), jnp.float32),
               jax.ShapeDtypeStruct((N, 1), jnp.float32)),
)


def _fused_body(hp, hrow, sdr, snr, sdT, snT, auxr, w1, w2, bcr, out_ref):
    eps = 1e-8
    a = auxr[...]
    an = jnp.maximum(jnp.sqrt(jnp.sum(a * a)), eps)
    sr = sdr[...] / (jnp.maximum(jnp.sqrt(snr[...]), eps) * an)
    sc = sdT[...] / (jnp.maximum(jnp.sqrt(snT[...]), eps) * an)
    acc = jnp.zeros((BR, D), jnp.float32)
    rs = jnp.zeros((BR, 1), jnp.float32)
    for t in range(NPAD // CB):
        dd = sc[:, t * CB:(t + 1) * CB] - sr
        kap = jnp.exp(-(dd * dd))
        acc = acc + jnp.dot(kap, hp[t * CB:(t + 1) * CB, :],
                            preferred_element_type=jnp.float32)
        rs = rs + jnp.sum(kap, axis=1, keepdims=True)
    z = acc / rs
    o = jnp.dot(hrow[...], w1[...], preferred_element_type=jnp.float32)
    o = o + jnp.dot(z, w2[...], preferred_element_type=jnp.float32)
    o = o + bcr[...]
    out_ref[...] = o


_fused = pl.pallas_call(
    _fused_body,
    grid=(NPAD // BR,),
    in_specs=[pl.BlockSpec((NPAD, D), lambda i: (0, 0)),
              pl.BlockSpec((BR, D), lambda i: (i, 0)),
              pl.BlockSpec((BR, 1), lambda i: (i, 0)),
              pl.BlockSpec((BR, 1), lambda i: (i, 0)),
              pl.BlockSpec((1, NPAD), lambda i: (0, 0)),
              pl.BlockSpec((1, NPAD), lambda i: (0, 0)),
              pl.BlockSpec((1, D), lambda i: (0, 0)),
              pl.BlockSpec((D, C), lambda i: (0, 0)),
              pl.BlockSpec((D, C), lambda i: (0, 0)),
              pl.BlockSpec((1, C), lambda i: (0, 0))],
    out_specs=pl.BlockSpec((BR, C), lambda i: (i, 0)),
    out_shape=jax.ShapeDtypeStruct((NPAD, C), jnp.float32),
)


def kernel(x, edge_index, W_l1, b_l1, W_r1, W_l2, b_l2, W_r2, aux, Wc, bc):
    src = edge_index[0].reshape(NW, NCH, K)
    dst = edge_index[1].reshape(NW, NCH, K)
    zfeat = jnp.zeros((RPT, D), jnp.float32)
    ones_h = jnp.ones((K, D), jnp.float32)

    psum1, pcnt = _get_sc_agg(True)(x, src, dst, zfeat, ones_h)
    h1 = _sage1(psum1, pcnt, x, W_l1.T, W_r1.T, b_l1.reshape(1, D))

    psum2 = _get_sc_agg(False)(h1, src, dst, zfeat, ones_h)
    h, hd, hn = _sage2(psum2, pcnt, h1, W_l2.T, W_r2.T, b_l2.reshape(1, D),
                       aux.reshape(D, 1))

    hp = jnp.pad(h, ((0, NPAD - N), (0, 0)))
    sdr = jnp.pad(hd, ((0, NPAD - N), (0, 0)), constant_values=1e30)
    snr = jnp.pad(hn, ((0, NPAD - N), (0, 0)), constant_values=1.0)
    out = _fused(hp, hp, sdr, snr, sdr.reshape(1, NPAD), snr.reshape(1, NPAD),
                 aux.reshape(1, D), Wc[:, :D].T, Wc[:, D:].T, bc.reshape(1, C))
    return out[:N]
